# Initial kernel scaffold; baseline (speedup 1.0000x reference)
import jax
import jax.numpy as jnp
from jax.experimental import pallas as pl


def kernel(species, energies, self_energies):
    raise NotImplementedError("write your pallas kernel here")


# speedup vs baseline: 373.6873x; 373.6873x over previous
import functools

import jax
import jax.numpy as jnp
from jax import lax
from jax.experimental import pallas as pl
from jax.experimental.pallas import tpu as pltpu
from jax.experimental.pallas import tpu_sc as plsc

_NC = 2
_NS = 16
_NW = _NC * _NS
_L = 16


@functools.lru_cache(maxsize=None)
def _sc_kernel(B, A):
    RPW = B // _NW
    G = 16
    NCH = RPW // G
    W = A * G
    NPAIR = G // 2
    NCHK = (2 * A) // _L
    CSTRAD = A // _L

    mesh = plsc.VectorSubcoreMesh(core_axis_name="c", subcore_axis_name="s")

    @functools.partial(
        pl.kernel,
        mesh=mesh,
        out_type=jax.ShapeDtypeStruct((B,), jnp.float32),
        compiler_params=pltpu.CompilerParams(needs_layout_passes=False),
        scratch_types=[
            pltpu.VMEM((W,), jnp.int32),
            pltpu.VMEM((RPW,), jnp.float32),
            pltpu.VMEM((RPW,), jnp.float32),
            pltpu.VMEM((_L,), jnp.float32),
        ],
    )
    def k(sp_hbm, en_hbm, tbl_hbm, out_hbm, sp_v, en_v, out_v, tbl_v):
        wid = lax.axis_index("s") * _NC + lax.axis_index("c")
        row0 = wid * RPW
        pltpu.sync_copy(tbl_hbm, tbl_v)
        pltpu.sync_copy(en_hbm.at[pl.ds(row0, RPW)], en_v)
        tbl = tbl_v[...]
        lanes = lax.iota(jnp.int32, _L)
        lo_half = lanes < (_L // 2)
        zero = jnp.zeros((_L,), jnp.float32)

        def chunk_body(ci, carry):
            pltpu.sync_copy(sp_hbm.at[pl.ds((row0 + ci * G) * A, W)], sp_v)
            res = zero
            for p in range(NPAIR):
                base = p * 2 * A
                acc0 = zero
                acc1 = zero
                for c in range(NCHK):
                    z = sp_v[pl.ds(base + c * _L, _L)]
                    zi = jnp.minimum(jnp.maximum(z, 0), _L - 1)
                    f = lax.gather(
                        tbl, zi[:, None],
                        lax.GatherDimensionNumbers(
                            offset_dims=(), collapsed_slice_dims=(0,),
                            start_index_map=(0,)),
                        (1,), mode=lax.GatherScatterMode.PROMISE_IN_BOUNDS)
                    if c < CSTRAD:
                        acc0 = acc0 + f
                    elif c == CSTRAD:
                        acc0 = acc0 + jnp.where(lo_half, f, zero)
                        acc1 = acc1 + jnp.where(lo_half, zero, f)
                    else:
                        acc1 = acc1 + f
                s0 = jnp.sum(acc0)
                s1 = jnp.sum(acc1)
                res = jnp.where(lanes == 2 * p, jnp.full((_L,), s0, jnp.float32), res)
                res = jnp.where(lanes == 2 * p + 1, jnp.full((_L,), s1, jnp.float32), res)
            out_v[pl.ds(ci * G, G)] = en_v[pl.ds(ci * G, G)] - res
            return carry

        lax.fori_loop(0, NCH, chunk_body, 0)
        pltpu.sync_copy(out_v, out_hbm.at[pl.ds(row0, RPW)])

    return k


def kernel(species, energies, self_energies):
    B, A = species.shape
    sp = jnp.asarray(species, jnp.int32).reshape(-1)
    tbl = jnp.zeros((_L,), jnp.float32).at[jnp.asarray([1, 6, 7, 8])].set(
        jnp.asarray(self_energies, jnp.float32))
    return _sc_kernel(B, A)(sp, jnp.asarray(energies, jnp.float32), tbl)


# speedup vs baseline: 448.2999x; 1.1997x over previous
import functools

import jax
import jax.numpy as jnp
from jax import lax
from jax.experimental import pallas as pl
from jax.experimental.pallas import tpu as pltpu
from jax.experimental.pallas import tpu_sc as plsc

_NC = 2
_NS = 16
_NW = _NC * _NS
_L = 16


@functools.lru_cache(maxsize=None)
def _sc_kernel(B, A):
    RPW = B // _NW
    G = 16
    NCH = RPW // G
    W = A * G
    NPAIR = G // 2
    NCHK = (2 * A) // _L
    CSTRAD = A // _L

    mesh = plsc.VectorSubcoreMesh(core_axis_name="c", subcore_axis_name="s")

    @functools.partial(
        pl.kernel,
        mesh=mesh,
        out_type=jax.ShapeDtypeStruct((B,), jnp.float32),
        compiler_params=pltpu.CompilerParams(needs_layout_passes=False),
        scratch_types=[
            pltpu.VMEM((W,), jnp.int32),
            pltpu.VMEM((W,), jnp.int32),
            pltpu.VMEM((RPW,), jnp.float32),
            pltpu.VMEM((RPW,), jnp.float32),
            pltpu.VMEM((_L,), jnp.float32),
            pltpu.SemaphoreType.DMA,
            pltpu.SemaphoreType.DMA,
        ],
    )
    def k(sp_hbm, en_hbm, tbl_hbm, out_hbm, sp_a, sp_b, en_v, out_v, tbl_v,
          sem_a, sem_b):
        wid = lax.axis_index("s") * _NC + lax.axis_index("c")
        row0 = wid * RPW
        pltpu.sync_copy(tbl_hbm, tbl_v)
        pltpu.sync_copy(en_hbm.at[pl.ds(row0, RPW)], en_v)
        tbl = tbl_v[...]
        lanes = lax.iota(jnp.int32, _L)
        lo_half = lanes < (_L // 2)
        zero = jnp.zeros((_L,), jnp.float32)
        dnums = lax.GatherDimensionNumbers(
            offset_dims=(), collapsed_slice_dims=(0,), start_index_map=(0,))

        def sp_off(ci):
            return (row0 + ci * G) * A

        def compute(sp_v, ci):
            res = zero
            for p in range(NPAIR):
                base = p * 2 * A
                acc0 = zero
                acc1 = zero
                for c in range(NCHK):
                    z = sp_v[pl.ds(base + c * _L, _L)]
                    zu = jnp.minimum(plsc.bitcast(z, jnp.uint32),
                                     jnp.uint32(_L - 1))
                    zi = plsc.bitcast(zu, jnp.int32)
                    f = lax.gather(
                        tbl, zi[:, None], dnums, (1,),
                        mode=lax.GatherScatterMode.PROMISE_IN_BOUNDS)
                    if c < CSTRAD:
                        acc0 = acc0 + f
                    elif c == CSTRAD:
                        acc0 = acc0 + jnp.where(lo_half, f, zero)
                        acc1 = acc1 + jnp.where(lo_half, zero, f)
                    else:
                        acc1 = acc1 + f
                s0 = jnp.sum(acc0)
                s1 = jnp.sum(acc1)
                res = jnp.where(lanes == 2 * p, jnp.full((_L,), s0, jnp.float32), res)
                res = jnp.where(lanes == 2 * p + 1, jnp.full((_L,), s1, jnp.float32), res)
            out_v[pl.ds(ci * G, G)] = en_v[pl.ds(ci * G, G)] - res

        pltpu.async_copy(sp_hbm.at[pl.ds(sp_off(0), W)], sp_a, sem_a)

        def body2(i, carry):
            c0 = 2 * i
            pltpu.async_copy(sp_hbm.at[pl.ds(sp_off(c0 + 1), W)], sp_b, sem_b)
            pltpu.make_async_copy(sp_hbm.at[pl.ds(0, W)], sp_a, sem_a).wait()
            compute(sp_a, c0)
            c2 = jnp.minimum(c0 + 2, NCH - 1)
            pltpu.async_copy(sp_hbm.at[pl.ds(sp_off(c2), W)], sp_a, sem_a)
            pltpu.make_async_copy(sp_hbm.at[pl.ds(0, W)], sp_b, sem_b).wait()
            compute(sp_b, c0 + 1)
            return carry

        lax.fori_loop(0, NCH // 2, body2, 0)
        pltpu.make_async_copy(sp_hbm.at[pl.ds(0, W)], sp_a, sem_a).wait()
        pltpu.sync_copy(out_v, out_hbm.at[pl.ds(row0, RPW)])

    return k


def kernel(species, energies, self_energies):
    B, A = species.shape
    sp = jnp.asarray(species, jnp.int32).reshape(-1)
    tbl = jnp.zeros((_L,), jnp.float32).at[jnp.asarray([1, 6, 7, 8])].set(
        jnp.asarray(self_energies, jnp.float32))
    return _sc_kernel(B, A)(sp, jnp.asarray(energies, jnp.float32), tbl)


# speedup vs baseline: 678.0585x; 1.5125x over previous
import functools

import jax
import jax.numpy as jnp
from jax import lax
from jax.experimental import pallas as pl
from jax.experimental.pallas import tpu as pltpu
from jax.experimental.pallas import tpu_sc as plsc

_NC = 2
_NS = 16
_NW = _NC * _NS
_L = 16
_Z = (1, 6, 7, 8)


@functools.lru_cache(maxsize=None)
def _sc_kernel(B, A):
    RPW = B // _NW
    G = 16
    NCH = RPW // G
    NFULL = A // _L
    REM = A - NFULL * _L

    mesh = plsc.VectorSubcoreMesh(core_axis_name="c", subcore_axis_name="s")

    @functools.partial(
        pl.kernel,
        mesh=mesh,
        out_type=jax.ShapeDtypeStruct((B,), jnp.float32),
        compiler_params=pltpu.CompilerParams(needs_layout_passes=False),
        scratch_types=[
            pltpu.VMEM((G, A), jnp.int32),
            pltpu.VMEM((G, A), jnp.int32),
            pltpu.VMEM((RPW,), jnp.float32),
            pltpu.VMEM((RPW,), jnp.float32),
            pltpu.VMEM((_L,), jnp.float32),
            pltpu.SemaphoreType.DMA,
            pltpu.SemaphoreType.DMA,
        ],
    )
    def k(sp_hbm, en_hbm, se_hbm, out_hbm, sp_a, sp_b, en_v, out_v, se_v,
          sem_a, sem_b):
        wid = lax.axis_index("s") * _NC + lax.axis_index("c")
        row0 = wid * RPW
        lanes = lax.iota(jnp.int32, _L)
        zero = jnp.zeros((_L,), jnp.float32)
        dnums = lax.GatherDimensionNumbers(
            offset_dims=(), collapsed_slice_dims=(0,), start_index_map=(0,))

        def vgather(vec, idx):
            return lax.gather(vec, idx[:, None], dnums, (1,),
                              mode=lax.GatherScatterMode.PROMISE_IN_BOUNDS)

        se_v[...] = zero
        pltpu.sync_copy(se_hbm, se_v.at[pl.ds(0, len(_Z))])
        nz = len(_Z)
        tmap = jnp.full((_L,), nz, jnp.int32)
        for i, z in enumerate(_Z):
            tmap = jnp.where(lanes == z, jnp.full((_L,), i, jnp.int32), tmap)
        tbl = vgather(se_v[...], tmap)

        pltpu.sync_copy(en_hbm.at[pl.ds(row0, RPW)], en_v)

        if REM:
            ov_mask = lanes >= (_L - REM)

        def lookup(z):
            zu = jnp.minimum(plsc.bitcast(z, jnp.uint32), jnp.uint32(_L - 1))
            return vgather(tbl, plsc.bitcast(zu, jnp.int32))

        def compute(sp_v, ci):
            res = zero
            for r in range(G):
                acc = zero
                for c in range(NFULL):
                    acc = acc + lookup(sp_v[r, pl.ds(c * _L, _L)])
                if REM:
                    f = lookup(sp_v[r, pl.ds(A - _L, _L)])
                    acc = acc + jnp.where(ov_mask, f, zero)
                s = jnp.sum(acc)
                res = jnp.where(lanes == r, jnp.full((_L,), s, jnp.float32), res)
            out_v[pl.ds(ci * G, G)] = en_v[pl.ds(ci * G, G)] - res

        def sp_src(ci):
            return sp_hbm.at[pl.ds(row0 + ci * G, G), :]

        pltpu.async_copy(sp_src(0), sp_a, sem_a)

        def body2(i, carry):
            c0 = 2 * i
            pltpu.async_copy(sp_src(c0 + 1), sp_b, sem_b)
            pltpu.make_async_copy(sp_src(0), sp_a, sem_a).wait()
            compute(sp_a, c0)
            c2 = jnp.minimum(c0 + 2, NCH - 1)
            pltpu.async_copy(sp_src(c2), sp_a, sem_a)
            pltpu.make_async_copy(sp_src(0), sp_b, sem_b).wait()
            compute(sp_b, c0 + 1)
            return carry

        lax.fori_loop(0, NCH // 2, body2, 0)
        pltpu.make_async_copy(sp_src(0), sp_a, sem_a).wait()
        pltpu.sync_copy(out_v, out_hbm.at[pl.ds(row0, RPW)])

    return k


def kernel(species, energies, self_energies):
    B, A = species.shape
    return _sc_kernel(B, A)(
        jnp.asarray(species, jnp.int32),
        jnp.asarray(energies, jnp.float32),
        jnp.asarray(self_energies, jnp.float32))


# speedup vs baseline: 1155.2695x; 1.7038x over previous
import functools

import jax
import jax.numpy as jnp
from jax import lax
from jax.experimental import pallas as pl
from jax.experimental.pallas import tpu as pltpu
from jax.experimental.pallas import tpu_sc as plsc

_NC = 2
_NS = 16
_NW = _NC * _NS
_L = 16
_Z = (1, 6, 7, 8)


@functools.lru_cache(maxsize=None)
def _sc_kernel(B, A):
    MPW = B // _NW
    MB = 128
    NCH = MPW // MB
    NG = MB // _L
    UNROLL = 2
    NFULL = A // UNROLL
    mesh = plsc.VectorSubcoreMesh(core_axis_name="c", subcore_axis_name="s")

    @functools.partial(
        pl.kernel,
        mesh=mesh,
        out_type=jax.ShapeDtypeStruct((B,), jnp.float32),
        compiler_params=pltpu.CompilerParams(needs_layout_passes=False),
        scratch_types=[
            pltpu.VMEM((A, MB), jnp.int32),
            pltpu.VMEM((A, MB), jnp.int32),
            pltpu.VMEM((MPW,), jnp.float32),
            pltpu.VMEM((MPW,), jnp.float32),
            pltpu.VMEM((_L,), jnp.float32),
            pltpu.SemaphoreType.DMA,
            pltpu.SemaphoreType.DMA,
        ],
    )
    def k(sp_hbm, en_hbm, se_hbm, out_hbm, sp_a, sp_b, en_v, out_v, se_v,
          sem_a, sem_b):
        wid = lax.axis_index("s") * _NC + lax.axis_index("c")
        col0 = wid * MPW
        lanes = lax.iota(jnp.int32, _L)
        zero = jnp.zeros((_L,), jnp.float32)
        dnums = lax.GatherDimensionNumbers(
            offset_dims=(), collapsed_slice_dims=(0,), start_index_map=(0,))

        def vgather(vec, idx):
            return lax.gather(vec, idx[:, None], dnums, (1,),
                              mode=lax.GatherScatterMode.PROMISE_IN_BOUNDS)

        se_v[...] = zero
        pltpu.sync_copy(se_hbm, se_v.at[pl.ds(0, len(_Z))])
        nz = len(_Z)
        tmap = jnp.full((_L,), nz, jnp.int32)
        for i, z in enumerate(_Z):
            tmap = jnp.where(lanes == z, jnp.full((_L,), i, jnp.int32), tmap)
        tbl = vgather(se_v[...], tmap)

        pltpu.sync_copy(en_hbm.at[pl.ds(col0, MPW)], en_v)

        def lookup(z):
            zu = jnp.minimum(plsc.bitcast(z, jnp.uint32), jnp.uint32(_L - 1))
            return vgather(tbl, plsc.bitcast(zu, jnp.int32))

        def compute(sp_v, ci):
            def abody(i, accs):
                accs = list(accs)
                for u in range(UNROLL):
                    a = i * UNROLL + u
                    for m in range(NG):
                        accs[m] = accs[m] + lookup(sp_v[a, pl.ds(m * _L, _L)])
                return tuple(accs)

            accs = lax.fori_loop(0, NFULL, abody, (zero,) * NG)
            accs = list(accs)
            for a in range(NFULL * UNROLL, A):
                for m in range(NG):
                    accs[m] = accs[m] + lookup(sp_v[a, pl.ds(m * _L, _L)])
            base = ci * MB
            for m in range(NG):
                sl = pl.ds(base + m * _L, _L)
                out_v[sl] = en_v[sl] - accs[m]

        def sp_src(ci):
            return sp_hbm.at[:, pl.ds(col0 + ci * MB, MB)]

        pltpu.async_copy(sp_src(0), sp_a, sem_a)

        def body2(i, carry):
            c0 = 2 * i
            pltpu.async_copy(sp_src(c0 + 1), sp_b, sem_b)
            pltpu.make_async_copy(sp_src(0), sp_a, sem_a).wait()
            compute(sp_a, c0)
            c2 = jnp.minimum(c0 + 2, NCH - 1)
            pltpu.async_copy(sp_src(c2), sp_a, sem_a)
            pltpu.make_async_copy(sp_src(0), sp_b, sem_b).wait()
            compute(sp_b, c0 + 1)
            return carry

        lax.fori_loop(0, NCH // 2, body2, 0)
        pltpu.make_async_copy(sp_src(0), sp_a, sem_a).wait()
        pltpu.sync_copy(out_v, out_hbm.at[pl.ds(col0, MPW)])

    return k


def kernel(species, energies, self_energies):
    B, A = species.shape
    return _sc_kernel(B, A)(
        jnp.asarray(species, jnp.int32).T,
        jnp.asarray(energies, jnp.float32),
        jnp.asarray(self_energies, jnp.float32))


# speedup vs baseline: 1159.2312x; 1.0034x over previous
import functools

import jax
import jax.numpy as jnp
from jax import lax
from jax.experimental import pallas as pl
from jax.experimental.pallas import tpu as pltpu
from jax.experimental.pallas import tpu_sc as plsc

_NC = 2
_NS = 16
_NW = _NC * _NS
_L = 16
_Z = (1, 6, 7, 8)


@functools.lru_cache(maxsize=None)
def _sc_kernel(B, A):
    MPW = B // _NW
    MB = 128
    NCH = MPW // MB
    NG = MB // _L
    UNROLL = 2
    NFULL = A // UNROLL
    mesh = plsc.VectorSubcoreMesh(core_axis_name="c", subcore_axis_name="s")

    @functools.partial(
        pl.kernel,
        mesh=mesh,
        out_type=jax.ShapeDtypeStruct((B,), jnp.float32),
        compiler_params=pltpu.CompilerParams(
            needs_layout_passes=False,
            skip_device_barrier=True,
            disable_bounds_checks=True,
        ),
        scratch_types=[
            pltpu.VMEM((A, MB), jnp.int32),
            pltpu.VMEM((A, MB), jnp.int32),
            pltpu.VMEM((MPW,), jnp.float32),
            pltpu.VMEM((MPW,), jnp.float32),
            pltpu.VMEM((_L,), jnp.float32),
            pltpu.SemaphoreType.DMA,
            pltpu.SemaphoreType.DMA,
        ],
    )
    def k(sp_hbm, en_hbm, se_hbm, out_hbm, sp_a, sp_b, en_v, out_v, se_v,
          sem_a, sem_b):
        wid = lax.axis_index("s") * _NC + lax.axis_index("c")
        col0 = wid * MPW
        lanes = lax.iota(jnp.int32, _L)
        zero = jnp.zeros((_L,), jnp.float32)
        dnums = lax.GatherDimensionNumbers(
            offset_dims=(), collapsed_slice_dims=(0,), start_index_map=(0,))

        def vgather(vec, idx):
            return lax.gather(vec, idx[:, None], dnums, (1,),
                              mode=lax.GatherScatterMode.PROMISE_IN_BOUNDS)

        se_v[...] = zero
        pltpu.sync_copy(se_hbm, se_v.at[pl.ds(0, len(_Z))])
        nz = len(_Z)
        tmap = jnp.full((_L,), nz, jnp.int32)
        for i, z in enumerate(_Z):
            tmap = jnp.where(lanes == z, jnp.full((_L,), i, jnp.int32), tmap)
        tbl = vgather(se_v[...], tmap)

        pltpu.sync_copy(en_hbm.at[pl.ds(col0, MPW)], en_v)

        def lookup(z):
            zu = jnp.minimum(plsc.bitcast(z, jnp.uint32), jnp.uint32(_L - 1))
            return vgather(tbl, plsc.bitcast(zu, jnp.int32))

        def compute(sp_v, ci):
            def abody(i, accs):
                accs = list(accs)
                for u in range(UNROLL):
                    a = i * UNROLL + u
                    for m in range(NG):
                        accs[m] = accs[m] + lookup(sp_v[a, pl.ds(m * _L, _L)])
                return tuple(accs)

            accs = lax.fori_loop(0, NFULL, abody, (zero,) * NG)
            accs = list(accs)
            for a in range(NFULL * UNROLL, A):
                for m in range(NG):
                    accs[m] = accs[m] + lookup(sp_v[a, pl.ds(m * _L, _L)])
            base = ci * MB
            for m in range(NG):
                sl = pl.ds(base + m * _L, _L)
                out_v[sl] = en_v[sl] - accs[m]

        def sp_src(ci):
            return sp_hbm.at[:, pl.ds(col0 + ci * MB, MB)]

        pltpu.async_copy(sp_src(0), sp_a, sem_a)

        def body2(i, carry):
            c0 = 2 * i
            pltpu.async_copy(sp_src(c0 + 1), sp_b, sem_b)
            pltpu.make_async_copy(sp_src(0), sp_a, sem_a).wait()
            compute(sp_a, c0)
            c2 = jnp.minimum(c0 + 2, NCH - 1)
            pltpu.async_copy(sp_src(c2), sp_a, sem_a)
            pltpu.make_async_copy(sp_src(0), sp_b, sem_b).wait()
            compute(sp_b, c0 + 1)
            return carry

        lax.fori_loop(0, NCH // 2, body2, 0)
        pltpu.make_async_copy(sp_src(0), sp_a, sem_a).wait()
        pltpu.sync_copy(out_v, out_hbm.at[pl.ds(col0, MPW)])

    return k


def kernel(species, energies, self_energies):
    B, A = species.shape
    return _sc_kernel(B, A)(
        jnp.asarray(species, jnp.int32).T,
        jnp.asarray(energies, jnp.float32),
        jnp.asarray(self_energies, jnp.float32))


# speedup vs baseline: 1180.1217x; 1.0180x over previous
import functools

import jax
import jax.numpy as jnp
from jax import lax
from jax.experimental import pallas as pl
from jax.experimental.pallas import tpu as pltpu
from jax.experimental.pallas import tpu_sc as plsc

_NC = 2
_NS = 16
_NW = _NC * _NS
_L = 16
_Z = (1, 6, 7, 8)


@functools.lru_cache(maxsize=None)
def _sc_kernel(B, A, B_sc):
    MPW = B_sc // _NW
    MB = 128
    NCH = MPW // MB
    NG = MB // _L
    UNROLL = 2
    NFULL = A // UNROLL
    mesh = plsc.VectorSubcoreMesh(core_axis_name="c", subcore_axis_name="s")

    @functools.partial(
        pl.kernel,
        mesh=mesh,
        out_type=jax.ShapeDtypeStruct((B_sc,), jnp.float32),
        compiler_params=pltpu.CompilerParams(
            needs_layout_passes=False,
            skip_device_barrier=True,
            disable_bounds_checks=True,
        ),
        scratch_types=[
            pltpu.VMEM((A, MB), jnp.int32),
            pltpu.VMEM((A, MB), jnp.int32),
            pltpu.VMEM((MPW,), jnp.float32),
            pltpu.VMEM((MPW,), jnp.float32),
            pltpu.VMEM((_L,), jnp.float32),
            pltpu.SemaphoreType.DMA,
            pltpu.SemaphoreType.DMA,
        ],
    )
    def k(sp_hbm, en_hbm, se_hbm, out_hbm, sp_a, sp_b, en_v, out_v, se_v,
          sem_a, sem_b):
        wid = lax.axis_index("s") * _NC + lax.axis_index("c")
        col0 = wid * MPW
        lanes = lax.iota(jnp.int32, _L)
        zero = jnp.zeros((_L,), jnp.float32)
        dnums = lax.GatherDimensionNumbers(
            offset_dims=(), collapsed_slice_dims=(0,), start_index_map=(0,))

        def vgather(vec, idx):
            return lax.gather(vec, idx[:, None], dnums, (1,),
                              mode=lax.GatherScatterMode.PROMISE_IN_BOUNDS)

        se_v[...] = zero
        pltpu.sync_copy(se_hbm, se_v.at[pl.ds(0, len(_Z))])
        nz = len(_Z)
        tmap = jnp.full((_L,), nz, jnp.int32)
        for i, z in enumerate(_Z):
            tmap = jnp.where(lanes == z, jnp.full((_L,), i, jnp.int32), tmap)
        tbl = vgather(se_v[...], tmap)

        pltpu.sync_copy(en_hbm.at[pl.ds(col0, MPW)], en_v)

        def lookup(z):
            zu = jnp.minimum(plsc.bitcast(z, jnp.uint32), jnp.uint32(_L - 1))
            return vgather(tbl, plsc.bitcast(zu, jnp.int32))

        def compute(sp_v, ci):
            def abody(i, accs):
                accs = list(accs)
                for u in range(UNROLL):
                    a = i * UNROLL + u
                    for m in range(NG):
                        accs[m] = accs[m] + lookup(sp_v[a, pl.ds(m * _L, _L)])
                return tuple(accs)

            accs = lax.fori_loop(0, NFULL, abody, (zero,) * NG)
            accs = list(accs)
            for a in range(NFULL * UNROLL, A):
                for m in range(NG):
                    accs[m] = accs[m] + lookup(sp_v[a, pl.ds(m * _L, _L)])
            base = ci * MB
            for m in range(NG):
                sl = pl.ds(base + m * _L, _L)
                out_v[sl] = en_v[sl] - accs[m]

        def sp_src(ci):
            return sp_hbm.at[:, pl.ds(col0 + ci * MB, MB)]

        pltpu.async_copy(sp_src(0), sp_a, sem_a)

        def body2(i, carry):
            c0 = 2 * i
            pltpu.async_copy(sp_src(c0 + 1), sp_b, sem_b)
            pltpu.make_async_copy(sp_src(0), sp_a, sem_a).wait()
            compute(sp_a, c0)
            c2 = jnp.minimum(c0 + 2, NCH - 1)
            pltpu.async_copy(sp_src(c2), sp_a, sem_a)
            pltpu.make_async_copy(sp_src(0), sp_b, sem_b).wait()
            compute(sp_b, c0 + 1)
            return carry

        lax.fori_loop(0, NCH // 2, body2, 0)
        pltpu.make_async_copy(sp_src(0), sp_a, sem_a).wait()
        pltpu.sync_copy(out_v, out_hbm.at[pl.ds(col0, MPW)])

    return k


@functools.lru_cache(maxsize=None)
def _tc_kernel(B, A, B_sc, BC=2048):
    B_tc = B - B_sc
    off = B_sc // BC

    def body(se_ref, sp_ref, en_ref, out_ref):
        z = sp_ref[...]
        tot = jnp.zeros(z.shape, jnp.float32)
        for i, zk in enumerate(_Z):
            tot = tot + jnp.where(z == zk, se_ref[i], jnp.float32(0.0))
        out_ref[...] = en_ref[...] - jnp.sum(tot, axis=0)

    return pl.pallas_call(
        body,
        grid=(B_tc // BC,),
        in_specs=[
            pl.BlockSpec(memory_space=pltpu.SMEM),
            pl.BlockSpec((A, BC), lambda i: (0, off + i)),
            pl.BlockSpec((BC,), lambda i: (off + i,)),
        ],
        out_specs=pl.BlockSpec((BC,), lambda i: (i,)),
        out_shape=jax.ShapeDtypeStruct((B_tc,), jnp.float32),
    )


def kernel(species, energies, self_energies):
    B, A = species.shape
    sp_t = jnp.asarray(species, jnp.int32).T
    en = jnp.asarray(energies, jnp.float32)
    se = jnp.asarray(self_energies, jnp.float32)
    B_sc = B // 2
    sc_out = _sc_kernel(B, A, B_sc)(sp_t, en, se)
    tc_out = _tc_kernel(B, A, B_sc)(se, sp_t, en)
    return jnp.concatenate([sc_out, tc_out])


# speedup vs baseline: 1192.1397x; 1.0102x over previous
import functools

import jax
import jax.numpy as jnp
from jax import lax
from jax.experimental import pallas as pl
from jax.experimental.pallas import tpu as pltpu
from jax.experimental.pallas import tpu_sc as plsc

_NC = 2
_NS = 16
_NW = _NC * _NS
_L = 16
_Z = (1, 6, 7, 8)


@functools.lru_cache(maxsize=None)
def _sc_kernel(B, A, B_sc):
    MPW = B_sc // _NW
    MB = 128
    NCH = MPW // MB
    NG = MB // _L
    UNROLL = 2
    NFULL = A // UNROLL
    mesh = plsc.VectorSubcoreMesh(core_axis_name="c", subcore_axis_name="s")

    @functools.partial(
        pl.kernel,
        mesh=mesh,
        out_type=jax.ShapeDtypeStruct((B_sc,), jnp.float32),
        compiler_params=pltpu.CompilerParams(
            needs_layout_passes=False,
            skip_device_barrier=True,
            disable_bounds_checks=True,
        ),
        scratch_types=[
            pltpu.VMEM((A, MB), jnp.int32),
            pltpu.VMEM((A, MB), jnp.int32),
            pltpu.VMEM((MPW,), jnp.float32),
            pltpu.VMEM((MPW,), jnp.float32),
            pltpu.VMEM((_L,), jnp.float32),
            pltpu.SemaphoreType.DMA,
            pltpu.SemaphoreType.DMA,
        ],
    )
    def k(sp_hbm, en_hbm, se_hbm, out_hbm, sp_a, sp_b, en_v, out_v, se_v,
          sem_a, sem_b):
        wid = lax.axis_index("s") * _NC + lax.axis_index("c")
        col0 = wid * MPW
        lanes = lax.iota(jnp.int32, _L)
        zero = jnp.zeros((_L,), jnp.float32)
        dnums = lax.GatherDimensionNumbers(
            offset_dims=(), collapsed_slice_dims=(0,), start_index_map=(0,))

        def vgather(vec, idx):
            return lax.gather(vec, idx[:, None], dnums, (1,),
                              mode=lax.GatherScatterMode.PROMISE_IN_BOUNDS)

        se_v[...] = zero
        pltpu.sync_copy(se_hbm, se_v.at[pl.ds(0, len(_Z))])
        nz = len(_Z)
        tmap = jnp.full((_L,), nz, jnp.int32)
        for i, z in enumerate(_Z):
            tmap = jnp.where(lanes == z, jnp.full((_L,), i, jnp.int32), tmap)
        tbl = vgather(se_v[...], tmap)

        pltpu.sync_copy(en_hbm.at[pl.ds(col0, MPW)], en_v)

        def lookup(z):
            zu = jnp.minimum(plsc.bitcast(z, jnp.uint32), jnp.uint32(_L - 1))
            return vgather(tbl, plsc.bitcast(zu, jnp.int32))

        def compute(sp_v, ci):
            def abody(i, accs):
                accs = list(accs)
                for u in range(UNROLL):
                    a = i * UNROLL + u
                    for m in range(NG):
                        accs[m] = accs[m] + lookup(sp_v[a, pl.ds(m * _L, _L)])
                return tuple(accs)

            accs = lax.fori_loop(0, NFULL, abody, (zero,) * NG)
            accs = list(accs)
            for a in range(NFULL * UNROLL, A):
                for m in range(NG):
                    accs[m] = accs[m] + lookup(sp_v[a, pl.ds(m * _L, _L)])
            base = ci * MB
            for m in range(NG):
                sl = pl.ds(base + m * _L, _L)
                out_v[sl] = en_v[sl] - accs[m]

        def sp_src(ci):
            return sp_hbm.at[:, pl.ds(col0 + ci * MB, MB)]

        pltpu.async_copy(sp_src(0), sp_a, sem_a)

        def body2(i, carry):
            c0 = 2 * i
            pltpu.async_copy(sp_src(c0 + 1), sp_b, sem_b)
            pltpu.make_async_copy(sp_src(0), sp_a, sem_a).wait()
            compute(sp_a, c0)
            c2 = jnp.minimum(c0 + 2, NCH - 1)
            pltpu.async_copy(sp_src(c2), sp_a, sem_a)
            pltpu.make_async_copy(sp_src(0), sp_b, sem_b).wait()
            compute(sp_b, c0 + 1)
            return carry

        lax.fori_loop(0, NCH // 2, body2, 0)
        pltpu.make_async_copy(sp_src(0), sp_a, sem_a).wait()
        pltpu.sync_copy(out_v, out_hbm.at[pl.ds(col0, MPW)])

    return k


@functools.lru_cache(maxsize=None)
def _tc_kernel(B, A, B_sc, BC=4096):
    B_tc = B - B_sc
    off = B_sc // BC

    def body(se_ref, sp_ref, en_ref, out_ref):
        z = sp_ref[...]
        tot = jnp.float32(0.0)
        for i, zk in enumerate(_Z):
            tot = jnp.where(z == zk, se_ref[i], tot)
        out_ref[...] = en_ref[...] - jnp.sum(tot, axis=0)

    return pl.pallas_call(
        body,
        grid=(B_tc // BC,),
        in_specs=[
            pl.BlockSpec(memory_space=pltpu.SMEM),
            pl.BlockSpec((A, BC), lambda i: (0, off + i)),
            pl.BlockSpec((BC,), lambda i: (off + i,)),
        ],
        out_specs=pl.BlockSpec((BC,), lambda i: (i,)),
        out_shape=jax.ShapeDtypeStruct((B_tc,), jnp.float32),
    )


def kernel(species, energies, self_energies):
    B, A = species.shape
    sp_t = jnp.asarray(species, jnp.int32).T
    en = jnp.asarray(energies, jnp.float32)
    se = jnp.asarray(self_energies, jnp.float32)
    B_sc = B // 2
    sc_out = _sc_kernel(B, A, B_sc)(sp_t, en, se)
    tc_out = _tc_kernel(B, A, B_sc)(se, sp_t, en)
    return jnp.concatenate([sc_out, tc_out])


# speedup vs baseline: 1292.7626x; 1.0844x over previous
import functools

import jax
import jax.numpy as jnp
from jax import lax
from jax.experimental import pallas as pl
from jax.experimental.pallas import tpu as pltpu
from jax.experimental.pallas import tpu_sc as plsc

_NC = 2
_NS = 16
_NW = _NC * _NS
_L = 16
_Z = (1, 6, 7, 8)


@functools.lru_cache(maxsize=None)
def _sc_kernel(B, A, B_sc):
    MPW = B_sc // _NW
    MB = 128
    NCH = MPW // MB
    NG = MB // _L
    UNROLL = 2
    NFULL = A // UNROLL
    mesh = plsc.VectorSubcoreMesh(core_axis_name="c", subcore_axis_name="s")

    @functools.partial(
        pl.kernel,
        mesh=mesh,
        out_type=jax.ShapeDtypeStruct((B_sc,), jnp.float32),
        compiler_params=pltpu.CompilerParams(
            needs_layout_passes=False,
            skip_device_barrier=True,
            disable_bounds_checks=True,
        ),
        scratch_types=[
            pltpu.VMEM((A, MB), jnp.int32),
            pltpu.VMEM((A, MB), jnp.int32),
            pltpu.VMEM((MPW,), jnp.float32),
            pltpu.VMEM((MPW,), jnp.float32),
            pltpu.VMEM((_L,), jnp.float32),
            pltpu.SemaphoreType.DMA,
            pltpu.SemaphoreType.DMA,
        ],
    )
    def k(sp_hbm, en_hbm, se_hbm, out_hbm, sp_a, sp_b, en_v, out_v, se_v,
          sem_a, sem_b):
        wid = lax.axis_index("s") * _NC + lax.axis_index("c")
        col0 = wid * MPW
        lanes = lax.iota(jnp.int32, _L)
        zero = jnp.zeros((_L,), jnp.float32)
        dnums = lax.GatherDimensionNumbers(
            offset_dims=(), collapsed_slice_dims=(0,), start_index_map=(0,))

        def vgather(vec, idx):
            return lax.gather(vec, idx[:, None], dnums, (1,),
                              mode=lax.GatherScatterMode.PROMISE_IN_BOUNDS)

        se_v[...] = zero
        pltpu.sync_copy(se_hbm, se_v.at[pl.ds(0, len(_Z))])
        nz = len(_Z)
        tmap = jnp.full((_L,), nz, jnp.int32)
        for i, z in enumerate(_Z):
            tmap = jnp.where(lanes == z, jnp.full((_L,), i, jnp.int32), tmap)
        tbl = vgather(se_v[...], tmap)

        pltpu.sync_copy(en_hbm.at[pl.ds(col0, MPW)], en_v)

        def lookup(z):
            zu = jnp.minimum(plsc.bitcast(z, jnp.uint32), jnp.uint32(_L - 1))
            return vgather(tbl, plsc.bitcast(zu, jnp.int32))

        def compute(sp_v, ci):
            def abody(i, accs):
                accs = list(accs)
                for u in range(UNROLL):
                    a = i * UNROLL + u
                    for m in range(NG):
                        accs[m] = accs[m] + lookup(sp_v[a, pl.ds(m * _L, _L)])
                return tuple(accs)

            accs = lax.fori_loop(0, NFULL, abody, (zero,) * NG)
            accs = list(accs)
            for a in range(NFULL * UNROLL, A):
                for m in range(NG):
                    accs[m] = accs[m] + lookup(sp_v[a, pl.ds(m * _L, _L)])
            base = ci * MB
            for m in range(NG):
                sl = pl.ds(base + m * _L, _L)
                out_v[sl] = en_v[sl] - accs[m]

        def sp_src(ci):
            return sp_hbm.at[:, pl.ds(col0 + ci * MB, MB)]

        bufs = ((sp_a, sem_a), (sp_b, sem_b))
        pltpu.async_copy(sp_src(0), sp_a, sem_a)
        for ci in range(NCH):
            buf, sem = bufs[ci % 2]
            if ci + 1 < NCH:
                nbuf, nsem = bufs[(ci + 1) % 2]
                pltpu.async_copy(sp_src(ci + 1), nbuf, nsem)
            pltpu.make_async_copy(sp_src(0), buf, sem).wait()
            compute(buf, ci)
        pltpu.sync_copy(out_v, out_hbm.at[pl.ds(col0, MPW)])

    return k


@functools.lru_cache(maxsize=None)
def _tc_kernel(B, A, B_sc, BC=4096):
    B_tc = B - B_sc
    off = B_sc // BC

    def body(se_ref, sp_ref, en_ref, out_ref):
        z = sp_ref[...]
        tot = jnp.float32(0.0)
        for i, zk in enumerate(_Z):
            tot = jnp.where(z == zk, se_ref[i], tot)
        out_ref[...] = en_ref[...] - jnp.sum(tot, axis=0)

    return pl.pallas_call(
        body,
        grid=(B_tc // BC,),
        in_specs=[
            pl.BlockSpec(memory_space=pltpu.SMEM),
            pl.BlockSpec((A, BC), lambda i: (0, off + i)),
            pl.BlockSpec((BC,), lambda i: (off + i,)),
        ],
        out_specs=pl.BlockSpec((BC,), lambda i: (i,)),
        out_shape=jax.ShapeDtypeStruct((B_tc,), jnp.float32),
    )


def kernel(species, energies, self_energies):
    B, A = species.shape
    sp_t = jnp.asarray(species, jnp.int32).T
    en = jnp.asarray(energies, jnp.float32)
    se = jnp.asarray(self_energies, jnp.float32)
    B_sc = B // 4
    sc_out = _sc_kernel(B, A, B_sc)(sp_t, en, se)
    tc_out = _tc_kernel(B, A, B_sc)(se, sp_t, en)
    return jnp.concatenate([sc_out, tc_out])


# speedup vs baseline: 1298.1266x; 1.0041x over previous
import functools

import jax
import jax.numpy as jnp
from jax import lax
from jax.experimental import pallas as pl
from jax.experimental.pallas import tpu as pltpu
from jax.experimental.pallas import tpu_sc as plsc

_NC = 2
_NS = 16
_NW = _NC * _NS
_L = 16
_Z = (1, 6, 7, 8)


@functools.lru_cache(maxsize=None)
def _sc_kernel(B, A, B_sc):
    MPW = B_sc // _NW
    MB = 128
    NCH = MPW // MB
    NG = MB // _L
    UNROLL = 2
    NFULL = A // UNROLL
    mesh = plsc.VectorSubcoreMesh(core_axis_name="c", subcore_axis_name="s")

    @functools.partial(
        pl.kernel,
        mesh=mesh,
        out_type=jax.ShapeDtypeStruct((B_sc,), jnp.float32),
        compiler_params=pltpu.CompilerParams(
            needs_layout_passes=False,
            skip_device_barrier=True,
            disable_bounds_checks=True,
        ),
        scratch_types=[
            pltpu.VMEM((A, MB), jnp.int32),
            pltpu.VMEM((A, MB), jnp.int32),
            pltpu.VMEM((MPW,), jnp.float32),
            pltpu.VMEM((MPW,), jnp.float32),
            pltpu.VMEM((_L,), jnp.float32),
            pltpu.SemaphoreType.DMA,
            pltpu.SemaphoreType.DMA,
        ],
    )
    def k(sp_hbm, en_hbm, se_hbm, out_hbm, sp_a, sp_b, en_v, out_v, se_v,
          sem_a, sem_b):
        wid = lax.axis_index("s") * _NC + lax.axis_index("c")
        col0 = wid * MPW
        lanes = lax.iota(jnp.int32, _L)
        zero = jnp.zeros((_L,), jnp.float32)
        dnums = lax.GatherDimensionNumbers(
            offset_dims=(), collapsed_slice_dims=(0,), start_index_map=(0,))

        def vgather(vec, idx):
            return lax.gather(vec, idx[:, None], dnums, (1,),
                              mode=lax.GatherScatterMode.PROMISE_IN_BOUNDS)

        se_v[...] = zero
        pltpu.sync_copy(se_hbm, se_v.at[pl.ds(0, len(_Z))])
        nz = len(_Z)
        tmap = jnp.full((_L,), nz, jnp.int32)
        for i, z in enumerate(_Z):
            tmap = jnp.where(lanes == z, jnp.full((_L,), i, jnp.int32), tmap)
        tbl = vgather(se_v[...], tmap)

        pltpu.sync_copy(en_hbm.at[pl.ds(col0, MPW)], en_v)

        def lookup(z):
            zu = jnp.minimum(plsc.bitcast(z, jnp.uint32), jnp.uint32(_L - 1))
            return vgather(tbl, plsc.bitcast(zu, jnp.int32))

        def compute(sp_v, ci):
            def abody(i, accs):
                accs = list(accs)
                for u in range(UNROLL):
                    a = i * UNROLL + u
                    for m in range(NG):
                        accs[m] = accs[m] + lookup(sp_v[a, pl.ds(m * _L, _L)])
                return tuple(accs)

            accs = lax.fori_loop(0, NFULL, abody, (zero,) * NG)
            accs = list(accs)
            for a in range(NFULL * UNROLL, A):
                for m in range(NG):
                    accs[m] = accs[m] + lookup(sp_v[a, pl.ds(m * _L, _L)])
            base = ci * MB
            for m in range(NG):
                sl = pl.ds(base + m * _L, _L)
                out_v[sl] = en_v[sl] - accs[m]

        def sp_src(ci):
            return sp_hbm.at[:, pl.ds(col0 + ci * MB, MB)]

        bufs = ((sp_a, sem_a), (sp_b, sem_b))
        pltpu.async_copy(sp_src(0), sp_a, sem_a)
        for ci in range(NCH):
            buf, sem = bufs[ci % 2]
            if ci + 1 < NCH:
                nbuf, nsem = bufs[(ci + 1) % 2]
                pltpu.async_copy(sp_src(ci + 1), nbuf, nsem)
            pltpu.make_async_copy(sp_src(0), buf, sem).wait()
            compute(buf, ci)
        pltpu.sync_copy(out_v, out_hbm.at[pl.ds(col0, MPW)])

    return k


@functools.lru_cache(maxsize=None)
def _tc_kernel(B, A, B_sc, BC=4096):
    B_tc = B - B_sc
    off = B_sc // BC

    def body(se_ref, sp_ref, en_ref, out_ref):
        z = sp_ref[...]
        tot = jnp.float32(0.0)
        for i, zk in enumerate(_Z):
            tot = jnp.where(z == zk, se_ref[i], tot)
        out_ref[...] = en_ref[...] - jnp.sum(tot, axis=0)

    return pl.pallas_call(
        body,
        grid=(B_tc // BC,),
        in_specs=[
            pl.BlockSpec(memory_space=pltpu.SMEM),
            pl.BlockSpec((A, BC), lambda i: (0, off + i)),
            pl.BlockSpec((BC,), lambda i: (off + i,)),
        ],
        out_specs=pl.BlockSpec((BC,), lambda i: (off + i,)),
        out_shape=jax.ShapeDtypeStruct((B,), jnp.float32),
    )


def kernel(species, energies, self_energies):
    B, A = species.shape
    sp_t = jnp.asarray(species, jnp.int32).T
    en = jnp.asarray(energies, jnp.float32)
    se = jnp.asarray(self_energies, jnp.float32)
    B_sc = B // 4
    sc_out = _sc_kernel(B, A, B_sc)(sp_t, en, se)
    tc_out = _tc_kernel(B, A, B_sc)(se, sp_t, en)
    return lax.dynamic_update_slice(tc_out, sc_out, (0,))


# speedup vs baseline: 1319.7907x; 1.0167x over previous
import functools

import jax
import jax.numpy as jnp
from jax import lax
from jax.experimental import pallas as pl
from jax.experimental.pallas import tpu as pltpu
from jax.experimental.pallas import tpu_sc as plsc

_NC = 2
_NS = 16
_NW = _NC * _NS
_L = 16
_Z = (1, 6, 7, 8)


@functools.lru_cache(maxsize=None)
def _sc_kernel(B, A, B_sc):
    MPW = B_sc // _NW
    MB = 128
    NCH = MPW // MB
    NG = MB // _L
    UNROLL = 2
    NFULL = A // UNROLL
    mesh = plsc.VectorSubcoreMesh(core_axis_name="c", subcore_axis_name="s")

    @functools.partial(
        pl.kernel,
        mesh=mesh,
        out_type=jax.ShapeDtypeStruct((B_sc,), jnp.float32),
        compiler_params=pltpu.CompilerParams(
            needs_layout_passes=False,
            skip_device_barrier=True,
            disable_bounds_checks=True,
        ),
        scratch_types=[
            pltpu.VMEM((A, MB), jnp.int32),
            pltpu.VMEM((A, MB), jnp.int32),
            pltpu.VMEM((MPW,), jnp.float32),
            pltpu.VMEM((MPW,), jnp.float32),
            pltpu.VMEM((_L,), jnp.float32),
            pltpu.SemaphoreType.DMA,
            pltpu.SemaphoreType.DMA,
        ],
    )
    def k(sp_hbm, en_hbm, se_hbm, out_hbm, sp_a, sp_b, en_v, out_v, se_v,
          sem_a, sem_b):
        wid = lax.axis_index("s") * _NC + lax.axis_index("c")
        col0 = wid * MPW
        lanes = lax.iota(jnp.int32, _L)
        zero = jnp.zeros((_L,), jnp.float32)
        dnums = lax.GatherDimensionNumbers(
            offset_dims=(), collapsed_slice_dims=(0,), start_index_map=(0,))

        def vgather(vec, idx):
            return lax.gather(vec, idx[:, None], dnums, (1,),
                              mode=lax.GatherScatterMode.PROMISE_IN_BOUNDS)

        se_v[...] = zero
        pltpu.sync_copy(se_hbm, se_v.at[pl.ds(0, len(_Z))])
        nz = len(_Z)
        tmap = jnp.full((_L,), nz, jnp.int32)
        for i, z in enumerate(_Z):
            tmap = jnp.where(lanes == z, jnp.full((_L,), i, jnp.int32), tmap)
        tbl = vgather(se_v[...], tmap)

        pltpu.sync_copy(en_hbm.at[pl.ds(col0, MPW)], en_v)

        def lookup(z):
            zu = jnp.minimum(plsc.bitcast(z, jnp.uint32), jnp.uint32(_L - 1))
            return vgather(tbl, plsc.bitcast(zu, jnp.int32))

        def compute(sp_v, ci):
            def abody(i, accs):
                accs = list(accs)
                for u in range(UNROLL):
                    a = i * UNROLL + u
                    for m in range(NG):
                        accs[m] = accs[m] + lookup(sp_v[a, pl.ds(m * _L, _L)])
                return tuple(accs)

            accs = lax.fori_loop(0, NFULL, abody, (zero,) * NG)
            accs = list(accs)
            for a in range(NFULL * UNROLL, A):
                for m in range(NG):
                    accs[m] = accs[m] + lookup(sp_v[a, pl.ds(m * _L, _L)])
            base = ci * MB
            for m in range(NG):
                sl = pl.ds(base + m * _L, _L)
                out_v[sl] = en_v[sl] - accs[m]

        def sp_src(ci):
            return sp_hbm.at[:, pl.ds(col0 + ci * MB, MB)]

        bufs = ((sp_a, sem_a), (sp_b, sem_b))
        pltpu.async_copy(sp_src(0), sp_a, sem_a)
        for ci in range(NCH):
            buf, sem = bufs[ci % 2]
            if ci + 1 < NCH:
                nbuf, nsem = bufs[(ci + 1) % 2]
                pltpu.async_copy(sp_src(ci + 1), nbuf, nsem)
            pltpu.make_async_copy(sp_src(0), buf, sem).wait()
            compute(buf, ci)
        pltpu.sync_copy(out_v, out_hbm.at[pl.ds(col0, MPW)])

    return k


@functools.lru_cache(maxsize=None)
def _tc_kernel(B, A, B_sc, BC=4096):
    B_tc = B - B_sc
    off = B_sc // BC

    def body(se_ref, sp_ref, en_ref, out_ref):
        z = sp_ref[...]
        tot = jnp.float32(0.0)
        for i, zk in enumerate(_Z):
            tot = jnp.where(z == zk, se_ref[i], tot)
        ones = jnp.ones((1, A), jnp.float32)
        s = lax.dot_general(ones, tot, (((1,), (0,)), ((), ())),
                            preferred_element_type=jnp.float32)
        out_ref[...] = en_ref[...] - s[0]

    return pl.pallas_call(
        body,
        grid=(B_tc // BC,),
        in_specs=[
            pl.BlockSpec(memory_space=pltpu.SMEM),
            pl.BlockSpec((A, BC), lambda i: (0, off + i)),
            pl.BlockSpec((BC,), lambda i: (off + i,)),
        ],
        out_specs=pl.BlockSpec((BC,), lambda i: (off + i,)),
        out_shape=jax.ShapeDtypeStruct((B,), jnp.float32),
    )


def kernel(species, energies, self_energies):
    B, A = species.shape
    sp_t = jnp.asarray(species, jnp.int32).T
    en = jnp.asarray(energies, jnp.float32)
    se = jnp.asarray(self_energies, jnp.float32)
    B_sc = B // 4
    sc_out = _sc_kernel(B, A, B_sc)(sp_t, en, se)
    tc_out = _tc_kernel(B, A, B_sc)(se, sp_t, en)
    return lax.dynamic_update_slice(tc_out, sc_out, (0,))
